# Initial kernel scaffold; baseline (speedup 1.0000x reference)
import jax
import jax.numpy as jnp
from jax.experimental import pallas as pl


def kernel(x, edge_index, sd_index, Wl1, bl1, Wr1, Wl2, bl2, Wr2, Wf1, bf1, Wf2, bf2):
    raise NotImplementedError("write your pallas kernel here")


# speedup vs baseline: 2.0245x; 2.0245x over previous
import jax
import jax.numpy as jnp
from jax import lax
from jax.experimental import pallas as pl
from jax.experimental.pallas import tpu as pltpu
from jax.experimental.pallas import tpu_sc as plsc

N = 10000
D = 128
E = 320000
SD = 100000

NC = 2
NS = 16
NW = NC * NS
L = 16

NP = 10240
RT = NP // NS
EP = 327680
ER = EP // 128
RW = ER // NW
HK = 2
HR = RW // HK
SDP = 102400
PW = SDP // NW
PC = PW // 128

_mesh = plsc.VectorSubcoreMesh(core_axis_name="c", subcore_axis_name="s")


def _zero_rows(ref, nrows, ncol):
    def row(i, carry):
        for j in range(ncol // L):
            ref[i, pl.ds(j * L, L)] = jnp.zeros((L,), jnp.float32)
        return carry
    lax.fori_loop(0, nrows, row, 0)


def _edge_agg_body(tbl, srcp, dstp, agg_out, agg_sh, src_v, dst_v, rows0, sem0):
    c = lax.axis_index("c")
    s = lax.axis_index("s")
    wid = s * NC + c

    _zero_rows(rows0, 128, D)
    for k in range(RT // 128):
        pltpu.sync_copy(rows0, agg_sh.at[pl.ds(s * RT + k * 128, 128)])
    plsc.subcore_barrier()

    for h in range(HK):
        base = wid * RW + h * HR
        pltpu.sync_copy(srcp.at[pl.ds(base, HR)], src_v)
        pltpu.sync_copy(dstp.at[pl.ds(base, HR)], dst_v)

        def step(j, carry):
            cp = pltpu.async_copy(tbl.at[src_v.at[j]], rows0, sem0)
            cp.wait()
            pltpu.sync_copy(rows0, agg_sh.at[dst_v.at[j]], add=True)
            return carry
        lax.fori_loop(0, HR, step, 0)

    plsc.subcore_barrier()

    for k in range(RT // 128):
        r0 = s * RT + k * 128
        pltpu.sync_copy(agg_sh.at[pl.ds(r0, 128)], rows0)
        pltpu.sync_copy(rows0, agg_out.at[c, pl.ds(r0, 128)])


_edge_agg = pl.kernel(
    _edge_agg_body,
    out_type=jax.ShapeDtypeStruct((NC, NP, D), jnp.float32),
    mesh=_mesh,
    scratch_types=[
        pltpu.VMEM_SHARED((NP, D), jnp.float32),
        pltpu.VMEM((HR, 128), jnp.int32),
        pltpu.VMEM((HR, 128), jnp.int32),
        pltpu.VMEM((128, D), jnp.float32),
        pltpu.SemaphoreType.DMA,
    ],
)


def _cnt_body(dstp, cnt_out, cnt_sh, dst_v, ones_v):
    c = lax.axis_index("c")
    s = lax.axis_index("s")
    wid = s * NC + c

    _zero_rows(ones_v, 128, L)
    for k in range(RT // 128):
        pltpu.sync_copy(ones_v, cnt_sh.at[pl.ds(s * RT + k * 128, 128)])

    def orow(i, carry):
        ones_v[i, pl.ds(0, L)] = jnp.ones((L,), jnp.float32)
        return carry
    lax.fori_loop(0, 128, orow, 0)
    plsc.subcore_barrier()

    for h in range(HK):
        base = wid * RW + h * HR
        pltpu.sync_copy(dstp.at[pl.ds(base, HR)], dst_v)

        def step(j, carry):
            pltpu.sync_copy(ones_v, cnt_sh.at[dst_v.at[j]], add=True)
            return carry
        lax.fori_loop(0, HR, step, 0)

    plsc.subcore_barrier()

    for k in range(RT // 128):
        r0 = s * RT + k * 128
        pltpu.sync_copy(cnt_sh.at[pl.ds(r0, 128)], ones_v)
        pltpu.sync_copy(ones_v, cnt_out.at[c, pl.ds(r0, 128)])


_cnt_kernel = pl.kernel(
    _cnt_body,
    out_type=jax.ShapeDtypeStruct((NC, NP, L), jnp.float32),
    mesh=_mesh,
    scratch_types=[
        pltpu.VMEM_SHARED((NP, L), jnp.float32),
        pltpu.VMEM((HR, 128), jnp.int32),
        pltpu.VMEM((128, L), jnp.float32),
    ],
)


def _pair_mlp_kernel(ta, tb, sidx, didx, wf2, bf2b, out,
                     sidx_v, didx_v, arows, brows, wf2_v, bf2_v, out_v,
                     sema, semb):
    c = lax.axis_index("c")
    s = lax.axis_index("s")
    wid = s * NC + c

    pltpu.sync_copy(wf2, wf2_v)
    pltpu.sync_copy(bf2b, bf2_v)
    pltpu.sync_copy(sidx.at[pl.ds(wid * PW, PW)], sidx_v)
    pltpu.sync_copy(didx.at[pl.ds(wid * PW, PW)], didx_v)

    lanes = lax.iota(jnp.int32, L)
    bf2s = bf2_v[pl.ds(0, L)][0]

    def chunk(j, carry):
        ca = pltpu.async_copy(ta.at[sidx_v.at[pl.ds(j * 128, 128)]], arows, sema)
        cb = pltpu.async_copy(tb.at[didx_v.at[pl.ds(j * 128, 128)]], brows, semb)
        ca.wait()
        cb.wait()

        def group(g, inner):
            pvec = lanes + g * L

            def feat(k, acc):
                wchunk = wf2_v[pl.ds(k * L, L)]
                for l in range(L):
                    colv = lanes * 0 + (k * L + l)
                    ga = plsc.load_gather(arows, [pvec, colv])
                    gb = plsc.load_gather(brows, [pvec, colv])
                    h = jnp.maximum(ga + gb, 0.0)
                    acc = acc + h * wchunk[l]
                return acc
            acc = lax.fori_loop(0, 2 * D // L, feat,
                                jnp.zeros((L,), jnp.float32))
            out_v[pl.ds(j * 128 + g * L, L)] = acc + bf2s
            return inner
        lax.fori_loop(0, 128 // L, group, 0)
        return carry
    lax.fori_loop(0, PC, chunk, 0)

    pltpu.sync_copy(out_v, out.at[pl.ds(wid * PW, PW)])


_pair_mlp = pl.kernel(
    _pair_mlp_kernel,
    out_type=jax.ShapeDtypeStruct((SDP,), jnp.float32),
    mesh=_mesh,
    scratch_types=[
        pltpu.VMEM((PW,), jnp.int32),
        pltpu.VMEM((PW,), jnp.int32),
        pltpu.VMEM((128, 2 * D), jnp.float32),
        pltpu.VMEM((128, 2 * D), jnp.float32),
        pltpu.VMEM((2 * D,), jnp.float32),
        pltpu.VMEM((L,), jnp.float32),
        pltpu.VMEM((PW,), jnp.float32),
        pltpu.SemaphoreType.DMA,
        pltpu.SemaphoreType.DMA,
    ],
    compiler_params=pltpu.CompilerParams(needs_layout_passes=False),
)


def _sage_dense_kernel(ap_ref, cp_ref, x_ref, wl_ref, bl_ref, wr_ref, o_ref):
    agg = ap_ref[0] + ap_ref[1]
    cnt = cp_ref[0, :, 0:1] + cp_ref[1, :, 0:1]
    mean = agg / jnp.maximum(cnt, 1.0)
    z = (jnp.dot(mean, wl_ref[...], preferred_element_type=jnp.float32)
         + jnp.dot(x_ref[...], wr_ref[...], preferred_element_type=jnp.float32)
         + bl_ref[...])
    o_ref[...] = jnp.maximum(z, 0.0)


def _tc_layer1(agg_parts, cnt_parts, x, wl, bl, wr):
    return pl.pallas_call(
        _sage_dense_kernel,
        grid=(NS,),
        in_specs=[
            pl.BlockSpec((NC, RT, D), lambda i: (0, i, 0)),
            pl.BlockSpec((NC, RT, L), lambda i: (0, i, 0)),
            pl.BlockSpec((RT, D), lambda i: (i, 0)),
            pl.BlockSpec((D, D), lambda i: (0, 0)),
            pl.BlockSpec((1, D), lambda i: (0, 0)),
            pl.BlockSpec((D, D), lambda i: (0, 0)),
        ],
        out_specs=pl.BlockSpec((RT, D), lambda i: (i, 0)),
        out_shape=jax.ShapeDtypeStruct((NP, D), jnp.float32),
    )(agg_parts, cnt_parts, x, wl, bl, wr)


def _sage2_pair_kernel(ap_ref, cp_ref, z1_ref, wl_ref, bl_ref, wr_ref,
                       u_ref, v_ref, bf1_ref, ta_ref, tb_ref):
    agg = ap_ref[0] + ap_ref[1]
    cnt = cp_ref[0, :, 0:1] + cp_ref[1, :, 0:1]
    mean = agg / jnp.maximum(cnt, 1.0)
    z2 = (jnp.dot(mean, wl_ref[...], preferred_element_type=jnp.float32)
          + jnp.dot(z1_ref[...], wr_ref[...], preferred_element_type=jnp.float32)
          + bl_ref[...])
    ta_ref[...] = jnp.dot(z2, u_ref[...], preferred_element_type=jnp.float32) + bf1_ref[...]
    tb_ref[...] = jnp.dot(z2, v_ref[...], preferred_element_type=jnp.float32)


def _tc_layer2(agg_parts, cnt_parts, z1, wl, bl, wr, u, v, bf1):
    return pl.pallas_call(
        _sage2_pair_kernel,
        grid=(NS,),
        in_specs=[
            pl.BlockSpec((NC, RT, D), lambda i: (0, i, 0)),
            pl.BlockSpec((NC, RT, L), lambda i: (0, i, 0)),
            pl.BlockSpec((RT, D), lambda i: (i, 0)),
            pl.BlockSpec((D, D), lambda i: (0, 0)),
            pl.BlockSpec((1, D), lambda i: (0, 0)),
            pl.BlockSpec((D, D), lambda i: (0, 0)),
            pl.BlockSpec((D, 2 * D), lambda i: (0, 0)),
            pl.BlockSpec((D, 2 * D), lambda i: (0, 0)),
            pl.BlockSpec((1, 2 * D), lambda i: (0, 0)),
        ],
        out_specs=[
            pl.BlockSpec((RT, 2 * D), lambda i: (i, 0)),
            pl.BlockSpec((RT, 2 * D), lambda i: (i, 0)),
        ],
        out_shape=[
            jax.ShapeDtypeStruct((NP, 2 * D), jnp.float32),
            jax.ShapeDtypeStruct((NP, 2 * D), jnp.float32),
        ],
    )(agg_parts, cnt_parts, z1, wl, bl, wr, u, v, bf1)


@jax.jit
def kernel(x, edge_index, sd_index, Wl1, bl1, Wr1, Wl2, bl2, Wr2, Wf1, bf1, Wf2, bf2):
    x_pad = jnp.zeros((NP, D), jnp.float32).at[:N].set(x)
    src = jnp.concatenate(
        [edge_index[0], jnp.zeros((EP - E,), jnp.int32)]).reshape(ER, 128)
    dst = jnp.concatenate(
        [edge_index[1], jnp.full((EP - E,), N, jnp.int32)]).reshape(ER, 128)

    cnt = _cnt_kernel(dst)
    agg1 = _edge_agg(x_pad, src, dst)
    z1 = _tc_layer1(agg1, cnt, x_pad, Wl1, bl1.reshape(1, D), Wr1)

    agg2 = _edge_agg(z1, src, dst)
    ta, tb = _tc_layer2(agg2, cnt, z1, Wl2, bl2.reshape(1, D), Wr2,
                        Wf1[:D], Wf1[D:], bf1.reshape(1, 2 * D))

    sidx = jnp.concatenate([sd_index[0], jnp.zeros((SDP - SD,), jnp.int32)])
    didx = jnp.concatenate([sd_index[1], jnp.zeros((SDP - SD,), jnp.int32)])
    out = _pair_mlp(ta, tb, sidx, didx, Wf2.reshape(2 * D),
                    jnp.broadcast_to(bf2, (L,)))
    return out[:SD]
